# baseline (speedup 1.0000x reference)
import functools

import jax
import jax.numpy as jnp
from jax import lax
from jax.experimental import pallas as pl
from jax.experimental.pallas import tpu as pltpu
from jax.experimental.pallas import tpu_sc as plsc

NC = 2
NS = 16
NW = NC * NS

D = 64
CG = 128
G = 1
C = CG * G
NBUF = 8


@functools.partial(jax.jit, static_argnums=(2,))
def _gather(weight, idx, B):
    b_per_w = B // NW
    n_chunks = b_per_w // C
    n_idx_rows = b_per_w // CG
    mesh = plsc.VectorSubcoreMesh(
        core_axis_name="c", subcore_axis_name="s", num_cores=NC, num_subcores=NS
    )

    @functools.partial(
        pl.kernel,
        out_type=jax.ShapeDtypeStruct((B, D), jnp.float32),
        mesh=mesh,
        scratch_types=[
            pltpu.VMEM((n_idx_rows, CG), jnp.int32),
            pltpu.VMEM((NBUF, C, D), jnp.float32),
            pltpu.SemaphoreType.DMA((NBUF,)),
            pltpu.SemaphoreType.DMA((NBUF,)),
        ],
        compiler_params=pltpu.CompilerParams(use_tc_tiling_on_sc=False),
    )
    def body(weight_hbm, idx_hbm, out_hbm, idx_v, bufs, gsem, wsem):
        wid = lax.axis_index("s") * NC + lax.axis_index("c")
        base = wid * b_per_w

        def gather_start(j, b):
            for g in range(G):
                pltpu.async_copy(
                    weight_hbm.at[idx_v.at[j * G + g]],
                    bufs.at[b, pl.ds(g * CG, CG)],
                    gsem.at[b],
                )

        def gather_wait(j, b):
            for g in range(G):
                pltpu.make_async_copy(
                    weight_hbm.at[idx_v.at[j * G + g]],
                    bufs.at[b, pl.ds(g * CG, CG)],
                    gsem.at[b],
                ).wait()

        def wb_start(j, b):
            pltpu.async_copy(
                bufs.at[b], out_hbm.at[pl.ds(base + j * C, C)], wsem.at[b]
            )

        def wb_wait(j, b):
            pltpu.make_async_copy(
                bufs.at[b], out_hbm.at[pl.ds(base + j * C, C)], wsem.at[b]
            ).wait()

        pltpu.sync_copy(idx_hbm.at[wid], idx_v)
        for b in range(NBUF):
            gather_start(b, b)

        @pl.loop(0, n_chunks - NBUF, step=NBUF)
        def _(j0):
            for b in range(NBUF):
                j = j0 + b
                gather_wait(j, b)
                wb_start(j, b)
                wb_wait(j, b)
                gather_start(j + NBUF, b)

        for b in range(NBUF):
            jlast = n_chunks - NBUF + b
            gather_wait(jlast, b)
            wb_start(jlast, b)
        for b in range(NBUF):
            jlast = n_chunks - NBUF + b
            wb_wait(jlast, b)

    idx3 = idx.reshape(NW, n_idx_rows, CG)
    return body(weight, idx3)


def kernel(input_, weight):
    Bm, S = input_.shape
    B = Bm * S
    idx = input_.reshape(B).astype(jnp.int32)
    out = _gather(weight, idx, B)
    return out.reshape(Bm, S, D)


# speedup vs baseline: 1.3468x; 1.3468x over previous
import functools

import jax
import jax.numpy as jnp
from jax import lax
from jax.experimental import pallas as pl
from jax.experimental.pallas import tpu as pltpu
from jax.experimental.pallas import tpu_sc as plsc

NC = 2
NS = 16
NW = NC * NS

D = 64
S_LOG = 50
S_PAD = 56
D_PAD = 128
NB = 2
NBUF = 8


@functools.partial(jax.jit, static_argnums=(2,))
def _gather(weight, idx, Bm):
    b_per_w = Bm // NW
    n_chunks = b_per_w // NB
    CI = NB * S_LOG
    mesh = plsc.VectorSubcoreMesh(
        core_axis_name="c", subcore_axis_name="s", num_cores=NC, num_subcores=NS
    )

    @functools.partial(
        pl.kernel,
        out_type=jax.ShapeDtypeStruct((Bm, S_PAD, D_PAD), jnp.float32),
        mesh=mesh,
        scratch_types=[
            pltpu.VMEM((n_chunks, CI), jnp.int32),
            pltpu.VMEM((NBUF, CI, D), jnp.float32),
            pltpu.SemaphoreType.DMA((NBUF,)),
            pltpu.SemaphoreType.DMA((NBUF,)),
        ],
        compiler_params=pltpu.CompilerParams(use_tc_tiling_on_sc=False),
    )
    def body(weight_hbm, idx_hbm, out_hbm, idx_v, bufs, gsem, wsem):
        wid = lax.axis_index("s") * NC + lax.axis_index("c")
        base = wid * b_per_w

        def gather_start(j, b):
            pltpu.async_copy(weight_hbm.at[idx_v.at[j]], bufs.at[b], gsem.at[b])

        def gather_wait(j, b):
            pltpu.make_async_copy(
                weight_hbm.at[idx_v.at[j]], bufs.at[b], gsem.at[b]
            ).wait()

        def wb_start(j, b):
            for k in range(NB):
                pltpu.async_copy(
                    bufs.at[b, pl.ds(k * S_LOG, S_LOG)],
                    out_hbm.at[base + j * NB + k, pl.ds(0, S_LOG), pl.ds(0, D)],
                    wsem.at[b],
                )

        def wb_wait(j, b):
            for k in range(NB):
                pltpu.make_async_copy(
                    bufs.at[b, pl.ds(k * S_LOG, S_LOG)],
                    out_hbm.at[base + j * NB + k, pl.ds(0, S_LOG), pl.ds(0, D)],
                    wsem.at[b],
                ).wait()

        pltpu.sync_copy(idx_hbm.at[wid], idx_v)
        for b in range(NBUF):
            gather_start(b, b)

        @pl.loop(0, n_chunks - NBUF, step=NBUF)
        def _(j0):
            for b in range(NBUF):
                j = j0 + b
                gather_wait(j, b)
                wb_start(j, b)
                wb_wait(j, b)
                gather_start(j + NBUF, b)

        for b in range(NBUF):
            jlast = n_chunks - NBUF + b
            gather_wait(jlast, b)
            wb_start(jlast, b)
        for b in range(NBUF):
            jlast = n_chunks - NBUF + b
            wb_wait(jlast, b)

    idx3 = idx.reshape(NW, n_chunks, CI)
    return body(weight, idx3)


def kernel(input_, weight):
    Bm, S = input_.shape
    idx = input_.reshape(Bm * S).astype(jnp.int32)
    out5 = _gather(weight, idx, Bm)
    return out5[:, :S_LOG, :D]
